# baseline (speedup 1.0000x reference)
import functools

import jax
import jax.numpy as jnp
from jax import lax
from jax.experimental import pallas as pl
from jax.experimental.pallas import tpu as pltpu
from jax.experimental.pallas import tpu_sc as plsc

_EPS = 1e-12
_LANES = 16
_CHUNK = 8
_NBUF = 4
_TBLK = 1024


def _make_sc_gather_sum(n_tok, hid):
    info = plsc.get_sparse_core_info()
    nw = info.num_cores * info.num_subcores
    per_w = n_tok // nw
    n_chunks = per_w // _CHUNK
    n_sl = hid // _LANES
    mesh = plsc.VectorSubcoreMesh(core_axis_name="c", subcore_axis_name="s")

    @functools.partial(
        pl.kernel,
        out_type=jax.ShapeDtypeStruct((n_tok, hid), jnp.float32),
        mesh=mesh,
        scratch_types=[
            pltpu.VMEM((per_w,), jnp.int32),
            pltpu.VMEM((per_w,), jnp.int32),
            [pltpu.VMEM((_CHUNK, hid), jnp.float32)] * _NBUF,
            [pltpu.VMEM((_CHUNK, hid), jnp.float32)] * _NBUF,
            [pltpu.VMEM((_CHUNK, hid), jnp.float32)] * 2,
            [pltpu.SemaphoreType.DMA] * _NBUF,
            [pltpu.SemaphoreType.DMA] * 2,
        ],
    )
    def k(aidx_h, bidx_h, word_h, out_h, ia, ib, bufa, bufb, obufs, sems,
          osems):
        wid = lax.axis_index("s") * info.num_cores + lax.axis_index("c")
        base = wid * per_w
        pltpu.sync_copy(aidx_h.at[pl.ds(base, per_w)], ia)
        pltpu.sync_copy(bidx_h.at[pl.ds(base, per_w)], ib)

        def prefetch(kk, slot):
            isl = pl.ds(kk * _CHUNK, _CHUNK)
            pltpu.async_copy(word_h.at[ia.at[isl]], bufa[slot], sems[slot])
            pltpu.async_copy(word_h.at[ib.at[isl]], bufb[slot], sems[slot])

        def wait_gather(slot):
            isl = pl.ds(0, _CHUNK)
            pltpu.make_async_copy(
                word_h.at[ia.at[isl]], bufa[slot], sems[slot]).wait()
            pltpu.make_async_copy(
                word_h.at[ib.at[isl]], bufb[slot], sems[slot]).wait()

        def out_wait(par):
            pltpu.make_async_copy(
                obufs[par], out_h.at[pl.ds(base, _CHUNK)], osems[par]).wait()

        def compute(kk, slot, par):
            ba, bb = bufa[slot], bufb[slot]
            ob = obufs[par]

            def tok_body(t, tc):
                for d in range(n_sl):
                    sl = pl.ds(d * _LANES, _LANES)
                    ob[t, sl] = ba[t, sl] + bb[t, sl]
                return tc

            lax.fori_loop(0, _CHUNK, tok_body, 0)
            pltpu.async_copy(
                ob, out_h.at[pl.ds(base + kk * _CHUNK, _CHUNK)], osems[par])

        for j in range(_NBUF - 1):
            prefetch(j, j)

        def ring_body(k4, carry):
            for j in range(_NBUF):
                kk = k4 * _NBUF + j
                wait_gather(j)

                @pl.when(kk + _NBUF - 1 < n_chunks)
                def _():
                    prefetch(kk + _NBUF - 1, (j + _NBUF - 1) % _NBUF)

                par = j % 2

                @pl.when(kk >= 2)
                def _():
                    out_wait(par)

                compute(kk, j, par)
            return carry

        lax.fori_loop(0, n_chunks // _NBUF, ring_body, 0)
        out_wait(0)
        out_wait(1)

    return k


def _tc_dense_body(g_ref, idx_ref, comb_ref, pos_ref, w_ref, b_ref, o_ref):
    idx = idx_ref[0, 0, :]
    n_comb = comb_ref.shape[0]
    hid = g_ref.shape[-1]
    oh = (idx[:, None] == lax.broadcasted_iota(jnp.int32, (_TBLK, n_comb), 1)
          ).astype(jnp.float32)
    crows = jnp.dot(oh, comb_ref[...], preferred_element_type=jnp.float32)
    x = g_ref[...] + pos_ref[...] + crows
    mu = jnp.mean(x, axis=-1, keepdims=True)
    var = jnp.mean(jnp.square(x - mu), axis=-1, keepdims=True)
    o_ref[...] = ((x - mu) * lax.rsqrt(var + _EPS) * w_ref[...]
                  + b_ref[...])


def kernel(input_ids, pos_ids, graph_rel, token_type_ids, word_emb, label_emb,
           pos_table, type_emb, ln_w, ln_b):
    b, s = input_ids.shape
    hid = word_emb.shape[1]
    aidx = input_ids.astype(jnp.int32)
    bidx = pos_ids.astype(jnp.int32)
    n_types = type_emb.shape[0]
    comb = (label_emb[:, None, :] + type_emb[None, :, :]).reshape(-1, hid)
    n_comb = comb.shape[0]
    comb_idx = (graph_rel * n_types + token_type_ids).astype(jnp.int32)

    sc_k = _make_sc_gather_sum(s, hid)
    n_sblk = s // _TBLK
    lnw2 = ln_w.reshape(1, hid)
    lnb2 = ln_b.reshape(1, hid)
    gs = [sc_k(aidx[p], bidx[p], word_emb) for p in range(b)]
    outs = []
    for p in range(b):
        idx3 = comb_idx[p].reshape(n_sblk, 1, _TBLK)
        out_p = pl.pallas_call(
            _tc_dense_body,
            grid=(n_sblk,),
            in_specs=[
                pl.BlockSpec((_TBLK, hid), lambda i: (i, 0)),
                pl.BlockSpec((1, 1, _TBLK), lambda i: (i, 0, 0)),
                pl.BlockSpec((n_comb, hid), lambda i: (0, 0)),
                pl.BlockSpec((_TBLK, hid), lambda i: (i, 0)),
                pl.BlockSpec((1, hid), lambda i: (0, 0)),
                pl.BlockSpec((1, hid), lambda i: (0, 0)),
            ],
            out_specs=pl.BlockSpec((_TBLK, hid), lambda i: (i, 0)),
            out_shape=jax.ShapeDtypeStruct((s, hid), jnp.float32),
        )(gs[p], idx3, comb, pos_table, lnw2, lnb2)
        outs.append(out_p)
    return jnp.stack(outs, axis=0)


# speedup vs baseline: 1.4158x; 1.4158x over previous
import functools

import jax
import jax.numpy as jnp
from jax import lax
from jax.experimental import pallas as pl
from jax.experimental.pallas import tpu as pltpu
from jax.experimental.pallas import tpu_sc as plsc

_EPS = 1e-12
_LANES = 16
_CHUNK = 8
_NBUF = 4
_TBLK = 1024


def _make_sc_gather_sum(n_tok, hid):
    info = plsc.get_sparse_core_info()
    nw = info.num_cores * info.num_subcores
    per_w = n_tok // nw
    n_chunks = per_w // _CHUNK
    n_sl = hid // _LANES
    mesh = plsc.VectorSubcoreMesh(core_axis_name="c", subcore_axis_name="s")

    @functools.partial(
        pl.kernel,
        out_type=jax.ShapeDtypeStruct((n_tok, hid), jnp.float32),
        mesh=mesh,
        scratch_types=[
            pltpu.VMEM((per_w,), jnp.int32),
            pltpu.VMEM((per_w,), jnp.int32),
            [pltpu.VMEM((_CHUNK, hid), jnp.float32)] * _NBUF,
            [pltpu.VMEM((_CHUNK, hid), jnp.float32)] * _NBUF,
            [pltpu.VMEM((_CHUNK, hid), jnp.float32)] * 2,
            [pltpu.SemaphoreType.DMA] * _NBUF,
            [pltpu.SemaphoreType.DMA] * 2,
        ],
    )
    def k(aidx_h, bidx_h, word_h, out_h, ia, ib, bufa, bufb, obufs, sems,
          osems):
        wid = lax.axis_index("s") * info.num_cores + lax.axis_index("c")
        base = wid * per_w
        pltpu.sync_copy(aidx_h.at[pl.ds(base, per_w)], ia)
        pltpu.sync_copy(bidx_h.at[pl.ds(base, per_w)], ib)

        def prefetch(kk, slot):
            isl = pl.ds(kk * _CHUNK, _CHUNK)
            pltpu.async_copy(word_h.at[ia.at[isl]], bufa[slot], sems[slot])
            pltpu.async_copy(word_h.at[ib.at[isl]], bufb[slot], sems[slot])

        def wait_gather(slot):
            isl = pl.ds(0, _CHUNK)
            pltpu.make_async_copy(
                word_h.at[ia.at[isl]], bufa[slot], sems[slot]).wait()
            pltpu.make_async_copy(
                word_h.at[ib.at[isl]], bufb[slot], sems[slot]).wait()

        def out_wait(par):
            pltpu.make_async_copy(
                obufs[par], out_h.at[pl.ds(base, _CHUNK)], osems[par]).wait()

        def compute(kk, slot, par):
            ba, bb = bufa[slot], bufb[slot]
            ob = obufs[par]

            def tok_body(t, tc):
                for d in range(n_sl):
                    sl = pl.ds(d * _LANES, _LANES)
                    ob[t, sl] = ba[t, sl] + bb[t, sl]
                return tc

            lax.fori_loop(0, _CHUNK, tok_body, 0)
            pltpu.async_copy(
                ob, out_h.at[pl.ds(base + kk * _CHUNK, _CHUNK)], osems[par])

        for j in range(_NBUF - 1):
            prefetch(j, j)

        def ring_body(k4, carry):
            for j in range(_NBUF):
                kk = k4 * _NBUF + j
                wait_gather(j)

                @pl.when(kk + _NBUF - 1 < n_chunks)
                def _():
                    prefetch(kk + _NBUF - 1, (j + _NBUF - 1) % _NBUF)

                par = j % 2

                @pl.when(kk >= 2)
                def _():
                    out_wait(par)

                compute(kk, j, par)
            return carry

        lax.fori_loop(0, n_chunks // _NBUF, ring_body, 0)
        out_wait(0)
        out_wait(1)

    return k


def _tc_dense_body(g_ref, idx_ref, comb_ref, pos_ref, w_ref, b_ref, o_ref):
    idx = idx_ref[0, 0, :]
    n_comb = comb_ref.shape[0]
    hid = g_ref.shape[-1]
    oh = (idx[:, None] == lax.broadcasted_iota(jnp.int32, (_TBLK, n_comb), 1)
          ).astype(jnp.float32)
    crows = jnp.dot(oh, comb_ref[...], preferred_element_type=jnp.float32)
    x = g_ref[...] + pos_ref[...] + crows
    ones_m = jnp.ones((hid, 128), jnp.float32)
    mu = jnp.dot(x, ones_m,
                 preferred_element_type=jnp.float32)[:, :1] * (1.0 / hid)
    ex2 = jnp.dot(x * x, ones_m,
                  preferred_element_type=jnp.float32)[:, :1] * (1.0 / hid)
    inv = lax.rsqrt(ex2 - mu * mu + _EPS)
    o_ref[...] = (x - mu) * inv * w_ref[...] + b_ref[...]


def kernel(input_ids, pos_ids, graph_rel, token_type_ids, word_emb, label_emb,
           pos_table, type_emb, ln_w, ln_b):
    b, s = input_ids.shape
    hid = word_emb.shape[1]
    aidx = input_ids.astype(jnp.int32)
    bidx = pos_ids.astype(jnp.int32)
    n_types = type_emb.shape[0]
    comb = (label_emb[:, None, :] + type_emb[None, :, :]).reshape(-1, hid)
    n_comb = comb.shape[0]
    comb_idx = (graph_rel * n_types + token_type_ids).astype(jnp.int32)

    n_tok = b * s
    g = _make_sc_gather_sum(n_tok, hid)(aidx.reshape(-1), bidx.reshape(-1),
                                        word_emb)

    n_blk = n_tok // _TBLK
    n_sblk = s // _TBLK
    idx3 = comb_idx.reshape(n_blk, 1, _TBLK)

    def tok_blk(i):
        return (i % b) * n_sblk + i // b

    out = pl.pallas_call(
        _tc_dense_body,
        grid=(n_blk,),
        in_specs=[
            pl.BlockSpec((_TBLK, hid), lambda i: (tok_blk(i), 0)),
            pl.BlockSpec((1, 1, _TBLK), lambda i: (tok_blk(i), 0, 0)),
            pl.BlockSpec((n_comb, hid), lambda i: (0, 0)),
            pl.BlockSpec((_TBLK, hid), lambda i: (i // b, 0)),
            pl.BlockSpec((1, hid), lambda i: (0, 0)),
            pl.BlockSpec((1, hid), lambda i: (0, 0)),
        ],
        out_specs=pl.BlockSpec((_TBLK, hid), lambda i: (tok_blk(i), 0)),
        out_shape=jax.ShapeDtypeStruct((n_tok, hid), jnp.float32),
    )(g, idx3, comb, pos_table, ln_w.reshape(1, hid), ln_b.reshape(1, hid))
    return out.reshape(b, s, hid)


# speedup vs baseline: 1.4735x; 1.0408x over previous
import functools

import jax
import jax.numpy as jnp
from jax import lax
from jax.experimental import pallas as pl
from jax.experimental.pallas import tpu as pltpu
from jax.experimental.pallas import tpu_sc as plsc

_EPS = 1e-12
_LANES = 16
_CHUNK = 8
_NBUF = 4
_TBLK = 2048


def _make_sc_gather_sum(n_tok, hid):
    info = plsc.get_sparse_core_info()
    nw = info.num_cores * info.num_subcores
    per_w = n_tok // nw
    n_chunks = per_w // _CHUNK
    n_sl = hid // _LANES
    mesh = plsc.VectorSubcoreMesh(core_axis_name="c", subcore_axis_name="s")

    @functools.partial(
        pl.kernel,
        out_type=jax.ShapeDtypeStruct((n_tok, hid), jnp.float32),
        mesh=mesh,
        scratch_types=[
            pltpu.VMEM((per_w,), jnp.int32),
            pltpu.VMEM((per_w,), jnp.int32),
            [pltpu.VMEM((_CHUNK, hid), jnp.float32)] * _NBUF,
            [pltpu.VMEM((_CHUNK, hid), jnp.float32)] * _NBUF,
            [pltpu.VMEM((_CHUNK, hid), jnp.float32)] * 2,
            [pltpu.SemaphoreType.DMA] * _NBUF,
            [pltpu.SemaphoreType.DMA] * 2,
        ],
    )
    def k(aidx_h, bidx_h, word_h, out_h, ia, ib, bufa, bufb, obufs, sems,
          osems):
        wid = lax.axis_index("s") * info.num_cores + lax.axis_index("c")
        base = wid * per_w
        pltpu.sync_copy(aidx_h.at[pl.ds(base, per_w)], ia)
        pltpu.sync_copy(bidx_h.at[pl.ds(base, per_w)], ib)

        def prefetch(kk, slot):
            isl = pl.ds(kk * _CHUNK, _CHUNK)
            pltpu.async_copy(word_h.at[ia.at[isl]], bufa[slot], sems[slot])
            pltpu.async_copy(word_h.at[ib.at[isl]], bufb[slot], sems[slot])

        def wait_gather(slot):
            isl = pl.ds(0, _CHUNK)
            pltpu.make_async_copy(
                word_h.at[ia.at[isl]], bufa[slot], sems[slot]).wait()
            pltpu.make_async_copy(
                word_h.at[ib.at[isl]], bufb[slot], sems[slot]).wait()

        def out_wait(par):
            pltpu.make_async_copy(
                obufs[par], out_h.at[pl.ds(base, _CHUNK)], osems[par]).wait()

        def compute(kk, slot, par):
            ba, bb = bufa[slot], bufb[slot]
            ob = obufs[par]

            def tok_body(t, tc):
                for d in range(n_sl):
                    sl = pl.ds(d * _LANES, _LANES)
                    ob[t, sl] = ba[t, sl] + bb[t, sl]
                return tc

            lax.fori_loop(0, _CHUNK, tok_body, 0)
            pltpu.async_copy(
                ob, out_h.at[pl.ds(base + kk * _CHUNK, _CHUNK)], osems[par])

        for j in range(_NBUF - 1):
            prefetch(j, j)

        def ring_body(k4, carry):
            for j in range(_NBUF):
                kk = k4 * _NBUF + j
                wait_gather(j)

                @pl.when(kk + _NBUF - 1 < n_chunks)
                def _():
                    prefetch(kk + _NBUF - 1, (j + _NBUF - 1) % _NBUF)

                par = j % 2

                @pl.when(kk >= 2)
                def _():
                    out_wait(par)

                compute(kk, j, par)
            return carry

        lax.fori_loop(0, n_chunks // _NBUF, ring_body, 0)
        out_wait(0)
        out_wait(1)

    return k


def _tc_dense_body(g_ref, idx_ref, comb_ref, pos_ref, w_ref, b_ref, o_ref):
    idx = idx_ref[0, 0, :]
    n_comb = comb_ref.shape[0]
    hid = g_ref.shape[-1]
    oh = (idx[:, None] == lax.broadcasted_iota(jnp.int32, (_TBLK, n_comb), 1)
          ).astype(jnp.float32)
    crows = jnp.dot(oh, comb_ref[...], preferred_element_type=jnp.float32)
    x = g_ref[...] + pos_ref[...] + crows
    mu = jnp.mean(x, axis=-1, keepdims=True)
    var = jnp.mean(jnp.square(x - mu), axis=-1, keepdims=True)
    o_ref[...] = ((x - mu) * lax.rsqrt(var + _EPS) * w_ref[...]
                  + b_ref[...])


def kernel(input_ids, pos_ids, graph_rel, token_type_ids, word_emb, label_emb,
           pos_table, type_emb, ln_w, ln_b):
    b, s = input_ids.shape
    hid = word_emb.shape[1]
    aidx = input_ids.astype(jnp.int32)
    bidx = pos_ids.astype(jnp.int32)
    n_types = type_emb.shape[0]
    comb = (label_emb[:, None, :] + type_emb[None, :, :]).reshape(-1, hid)
    n_comb = comb.shape[0]
    comb_idx = (graph_rel * n_types + token_type_ids).astype(jnp.int32)

    n_tok = b * s
    g = _make_sc_gather_sum(n_tok, hid)(aidx.reshape(-1), bidx.reshape(-1),
                                        word_emb)

    n_blk = n_tok // _TBLK
    n_sblk = s // _TBLK
    idx3 = comb_idx.reshape(n_blk, 1, _TBLK)

    def tok_blk(i):
        return (i % b) * n_sblk + i // b

    out = pl.pallas_call(
        _tc_dense_body,
        grid=(n_blk,),
        in_specs=[
            pl.BlockSpec((_TBLK, hid), lambda i: (tok_blk(i), 0)),
            pl.BlockSpec((1, 1, _TBLK), lambda i: (tok_blk(i), 0, 0)),
            pl.BlockSpec((n_comb, hid), lambda i: (0, 0)),
            pl.BlockSpec((_TBLK, hid), lambda i: (i // b, 0)),
            pl.BlockSpec((1, hid), lambda i: (0, 0)),
            pl.BlockSpec((1, hid), lambda i: (0, 0)),
        ],
        out_specs=pl.BlockSpec((_TBLK, hid), lambda i: (tok_blk(i), 0)),
        out_shape=jax.ShapeDtypeStruct((n_tok, hid), jnp.float32),
    )(g, idx3, comb, pos_table, ln_w.reshape(1, hid), ln_b.reshape(1, hid))
    return out.reshape(b, s, hid)
